# baseline (speedup 1.0000x reference)
import functools

import numpy as np
import jax
import jax.numpy as jnp
from jax import lax
from jax.experimental import pallas as pl
from jax.experimental.pallas import tpu as pltpu
from jax.experimental.pallas import tpu_sc as plsc

B = 64
N = 13860
H = W = 512
NPAD = 14336
NB = 2048
HW = H * W
NQ = 4
QSIZE = HW // NQ
SENT = -(1 << 20)
NC, NS = 2, 16
NWORK = NC * NS
IMGS_PER_WORKER = B // NWORK

def _bf16_round(x):
    u = lax.bitcast_convert_type(x, jnp.uint32)
    r = (u + jnp.uint32(0x7FFF) + ((u >> 16) & jnp.uint32(1))) & jnp.uint32(0xFFFF0000)
    return lax.bitcast_convert_type(r, jnp.float32)


def _proj_body(v_ref, p_ref, pts_ref, idx_ref):
    j = pl.program_id(0)
    Vm = _bf16_round(v_ref[:])
    Pm = _bf16_round(p_ref[:])
    rows = []
    for i in (0, 1, 3):
        acc = None
        for k in range(4):
            term = Pm[:, 4 * i + k:4 * i + k + 1] * Vm[:, 4 * k:4 * k + 4]
            acc = term if acc is None else acc + term
        rows.append(_bf16_round(acc))
    vpx, vpy, vpw = rows
    px = _bf16_round(pts_ref[0:1, :])
    py = _bf16_round(pts_ref[1:2, :])
    pz = _bf16_round(pts_ref[2:3, :])

    def proj(c):
        return c[:, 0:1] * px + c[:, 1:2] * py + c[:, 2:3] * pz + c[:, 3:4]

    tx = proj(vpx)
    ty = proj(vpy)
    tw = proj(vpw)
    nz = tw != 0.0
    X = jnp.where(nz, tx / tw, tx)
    Y = jnp.where(nz, ty / tw, ty)
    xs = (X + 1.0) * 0.5 * float(W)
    ys = (1.0 - (Y + 1.0) * 0.5) * float(H)
    xr = jnp.round(xs)
    yr = jnp.round(ys)
    valid = (xr >= 0.0) & (xr <= float(W - 1)) & (yr >= 0.0) & (yr <= float(H - 1))
    col = j * NB + lax.broadcasted_iota(jnp.int32, (B, NB), 1)
    valid = valid & (col < N)
    flat = yr.astype(jnp.int32) * W + xr.astype(jnp.int32)
    idx_ref[:] = jnp.where(valid, flat, SENT)


_proj_call = pl.pallas_call(
    _proj_body,
    grid=(NPAD // NB,),
    in_specs=[
        pl.BlockSpec((B, 16), lambda j: (0, 0)),
        pl.BlockSpec((B, 16), lambda j: (0, 0)),
        pl.BlockSpec((8, NB), lambda j: (0, j)),
    ],
    out_specs=pl.BlockSpec((B, NB), lambda j: (0, j)),
    out_shape=jax.ShapeDtypeStruct((B, NPAD), jnp.int32),
)


NR = 8
RROWS = H // NR
RSIZE = RROWS * W


def _sc_scatter_body(idx_hbm, out_hbm, idx_v, buf0, buf1, sem0, sem1):
    wid = lax.axis_index("s") * NC + lax.axis_index("c")
    zeros16 = jnp.zeros((16,), jnp.float32)
    v255 = jnp.full((16,), 255.0, jnp.float32)
    bufs = (buf0, buf1)
    sems = (sem0, sem1)
    pending = [None, None]
    cnt = 0
    for im in range(IMGS_PER_WORKER):
        b = wid * IMGS_PER_WORKER + im
        pltpu.sync_copy(idx_hbm.at[b], idx_v)
        for e in range(NR):
            k = cnt & 1
            buf = bufs[k]
            if pending[k] is not None:
                pending[k].wait()

            @pl.loop(0, RROWS)
            def zero_body(i, buf=buf):
                for cc in range(W // 16):
                    buf[i, pl.ds(cc * 16, 16)] = zeros16

            lo = e * RSIZE

            @pl.loop(0, NPAD // 16, unroll=8)
            def scan_body(i, lo=lo, buf=buf):
                v = idx_v[pl.ds(i * 16, 16)]
                u = v - lo
                m = lax.bitcast_convert_type(u, jnp.uint32) < jnp.uint32(RSIZE)
                row = u >> 9
                col = u & (W - 1)
                plsc.store_scatter(buf, [row, col], v255, mask=m)

            pending[k] = pltpu.async_copy(
                buf, out_hbm.at[b, pl.ds(e * RROWS, RROWS)], sems[k])
            cnt += 1
    for k in range(2):
        if pending[k] is not None:
            pending[k].wait()


@functools.lru_cache(maxsize=1)
def _sc_scatter_call():
    mesh = plsc.VectorSubcoreMesh(
        core_axis_name="c", subcore_axis_name="s",
        num_cores=NC, num_subcores=NS)
    return pl.kernel(
        _sc_scatter_body,
        out_type=jax.ShapeDtypeStruct((B, H, W), jnp.float32),
        mesh=mesh,
        compiler_params=pltpu.CompilerParams(needs_layout_passes=False),
        scratch_types=[
            pltpu.VMEM((NPAD,), jnp.int32),
            pltpu.VMEM((RROWS, W), jnp.float32),
            pltpu.VMEM((RROWS, W), jnp.float32),
            pltpu.SemaphoreType.DMA,
            pltpu.SemaphoreType.DMA,
        ],
    )


def _gauss_weights():
    sigma = 0.3 * ((9 - 1) * 0.5 - 1.0) + 0.8
    x = np.arange(9, dtype=np.float32) - 4.0
    k = np.exp(-(x.astype(np.float32) ** 2) / np.float32(2.0 * sigma * sigma))
    k = k.astype(np.float32)
    k = k / k.sum(dtype=np.float32)
    return [float(v) for v in k]


_GW = _gauss_weights()


def _band_ones():
    i = np.arange(H)
    return (np.abs(i[:, None] - i[None, :]) <= 4).astype(np.float32)


def _gauss_op():
    g = np.zeros((H, H), np.float32)
    gw = np.asarray(_GW, np.float32)
    for i in range(H):
        for k in range(9):
            t = i + k - 4
            if t < 0:
                t = -t
            elif t > H - 1:
                t = 2 * (H - 1) - t
            g[i, t] += gw[k]
    return g


def _morph_body(bo_ref, gg_ref, ggt_ref, r_ref, o_ref):
    m = r_ref[0]
    bo = bo_ref[:]
    c = jnp.dot(bo, m)
    c2 = jnp.dot(c, bo)
    d = jnp.where(c2 > 0.5, 255.0, 0.0)
    hp = jax.lax.Precision.HIGHEST
    s1 = jnp.dot(gg_ref[:], d, precision=hp)
    s = jnp.dot(s1, ggt_ref[:], precision=hp)
    ob = jnp.where(s > 100.0, 1.0, 0.0)
    o_ref[0, 0] = ob
    o_ref[0, 1] = ob
    o_ref[0, 2] = ob


_morph_call = pl.pallas_call(
    _morph_body,
    grid=(B,),
    in_specs=[
        pl.BlockSpec((H, H), lambda b: (0, 0)),
        pl.BlockSpec((H, H), lambda b: (0, 0)),
        pl.BlockSpec((H, H), lambda b: (0, 0)),
        pl.BlockSpec((1, H, W), lambda b: (b, 0, 0)),
    ],
    out_specs=pl.BlockSpec((1, 3, H, W), lambda b: (b, 0, 0, 0)),
    out_shape=jax.ShapeDtypeStruct((B, 3, H, W), jnp.float32),
)


def kernel(V_matrix, P_matrix, raw_base_points):
    V16 = V_matrix.reshape(B, 16)
    P16 = P_matrix.reshape(B, 16)
    ptsT = jnp.zeros((8, NPAD), jnp.float32)
    ptsT = ptsT.at[0:3, 0:N].set(raw_base_points[:, 0:3].T)
    idx = _proj_call(V16, P16, ptsT)
    raster = _sc_scatter_call()(idx)
    bo = jnp.asarray(_band_ones())
    gg = jnp.asarray(_gauss_op())
    img = _morph_call(bo, gg, gg.T, raster)
    return img


# speedup vs baseline: 1.1116x; 1.1116x over previous
import functools

import numpy as np
import jax
import jax.numpy as jnp
from jax import lax
from jax.experimental import pallas as pl
from jax.experimental.pallas import tpu as pltpu
from jax.experimental.pallas import tpu_sc as plsc

B = 64
N = 13860
H = W = 512
NPAD = 14336
NB = 2048
HW = H * W
NQ = 4
QSIZE = HW // NQ
SENT = -(1 << 20)
NC, NS = 2, 16
NWORK = NC * NS
IMGS_PER_WORKER = B // NWORK

def _bf16_round(x):
    u = lax.bitcast_convert_type(x, jnp.uint32)
    r = (u + jnp.uint32(0x7FFF) + ((u >> 16) & jnp.uint32(1))) & jnp.uint32(0xFFFF0000)
    return lax.bitcast_convert_type(r, jnp.float32)


def _proj_body(v_ref, p_ref, pts_ref, idx_ref):
    j = pl.program_id(0)
    Vm = _bf16_round(v_ref[:])
    Pm = _bf16_round(p_ref[:])
    rows = []
    for i in (0, 1, 3):
        acc = None
        for k in range(4):
            term = Pm[:, 4 * i + k:4 * i + k + 1] * Vm[:, 4 * k:4 * k + 4]
            acc = term if acc is None else acc + term
        rows.append(_bf16_round(acc))
    vpx, vpy, vpw = rows
    px = _bf16_round(pts_ref[0:1, :])
    py = _bf16_round(pts_ref[1:2, :])
    pz = _bf16_round(pts_ref[2:3, :])

    def proj(c):
        return c[:, 0:1] * px + c[:, 1:2] * py + c[:, 2:3] * pz + c[:, 3:4]

    tx = proj(vpx)
    ty = proj(vpy)
    tw = proj(vpw)
    nz = tw != 0.0
    X = jnp.where(nz, tx / tw, tx)
    Y = jnp.where(nz, ty / tw, ty)
    xs = (X + 1.0) * 0.5 * float(W)
    ys = (1.0 - (Y + 1.0) * 0.5) * float(H)
    xr = jnp.round(xs)
    yr = jnp.round(ys)
    valid = (xr >= 0.0) & (xr <= float(W - 1)) & (yr >= 0.0) & (yr <= float(H - 1))
    col = j * NB + lax.broadcasted_iota(jnp.int32, (B, NB), 1)
    valid = valid & (col < N)
    flat = yr.astype(jnp.int32) * W + xr.astype(jnp.int32)
    idx_ref[:] = jnp.where(valid, flat, SENT)


_proj_call = pl.pallas_call(
    _proj_body,
    grid=(NPAD // NB,),
    in_specs=[
        pl.BlockSpec((B, 16), lambda j: (0, 0)),
        pl.BlockSpec((B, 16), lambda j: (0, 0)),
        pl.BlockSpec((8, NB), lambda j: (0, j)),
    ],
    out_specs=pl.BlockSpec((B, NB), lambda j: (0, j)),
    out_shape=jax.ShapeDtypeStruct((B, NPAD), jnp.int32),
)


QROWS = H // NQ


def _sc_scatter_body(idx_hbm, out_hbm, idx_v, qbuf):
    wid = lax.axis_index("s") * NC + lax.axis_index("c")
    zeros16 = jnp.zeros((16,), jnp.float32)
    v255 = jnp.full((16,), 255.0, jnp.float32)
    for im in range(IMGS_PER_WORKER):
        b = wid * IMGS_PER_WORKER + im
        pltpu.sync_copy(idx_hbm.at[b], idx_v)
        for q in range(NQ):
            lo = q * QSIZE

            @pl.loop(0, QROWS)
            def zero_body(i):
                for cc in range(W // 16):
                    qbuf[i, pl.ds(cc * 16, 16)] = zeros16

            @pl.loop(0, NPAD // 16, unroll=8)
            def scan_body(i, lo=lo):
                v = idx_v[pl.ds(i * 16, 16)]
                u = v - lo
                m = lax.bitcast_convert_type(u, jnp.uint32) < jnp.uint32(QSIZE)
                row = u >> 9
                col = u & (W - 1)
                plsc.store_scatter(qbuf, [row, col], v255, mask=m)

            pltpu.sync_copy(qbuf, out_hbm.at[b, pl.ds(q * QROWS, QROWS)])


@functools.lru_cache(maxsize=1)
def _sc_scatter_call():
    mesh = plsc.VectorSubcoreMesh(
        core_axis_name="c", subcore_axis_name="s",
        num_cores=NC, num_subcores=NS)
    return pl.kernel(
        _sc_scatter_body,
        out_type=jax.ShapeDtypeStruct((B, H, W), jnp.float32),
        mesh=mesh,
        compiler_params=pltpu.CompilerParams(needs_layout_passes=False),
        scratch_types=[
            pltpu.VMEM((NPAD,), jnp.int32),
            pltpu.VMEM((QROWS, W), jnp.float32),
        ],
    )


def _gauss_weights():
    sigma = 0.3 * ((9 - 1) * 0.5 - 1.0) + 0.8
    x = np.arange(9, dtype=np.float32) - 4.0
    k = np.exp(-(x.astype(np.float32) ** 2) / np.float32(2.0 * sigma * sigma))
    k = k.astype(np.float32)
    k = k / k.sum(dtype=np.float32)
    return [float(v) for v in k]


_GW = _gauss_weights()


def _band_ones():
    i = np.arange(H)
    return (np.abs(i[:, None] - i[None, :]) <= 4).astype(np.float32)


def _gauss_op():
    g = np.zeros((H, H), np.float32)
    gw = np.asarray(_GW, np.float32)
    for i in range(H):
        for k in range(9):
            t = i + k - 4
            if t < 0:
                t = -t
            elif t > H - 1:
                t = 2 * (H - 1) - t
            g[i, t] += gw[k]
    return g


def _morph_body(bo_ref, gg_ref, ggt_ref, r_ref, o_ref):
    m = r_ref[0]
    bo = bo_ref[:]
    c = jnp.dot(bo, m)
    c2 = jnp.dot(c, bo)
    d = jnp.where(c2 > 0.5, 255.0, 0.0)
    hp = jax.lax.Precision.HIGHEST
    s1 = jnp.dot(gg_ref[:], d, precision=hp)
    s = jnp.dot(s1, ggt_ref[:], precision=hp)
    ob = jnp.where(s > 100.0, 1.0, 0.0)
    o_ref[0, 0] = ob
    o_ref[0, 1] = ob
    o_ref[0, 2] = ob


_morph_call = pl.pallas_call(
    _morph_body,
    grid=(B,),
    in_specs=[
        pl.BlockSpec((H, H), lambda b: (0, 0)),
        pl.BlockSpec((H, H), lambda b: (0, 0)),
        pl.BlockSpec((H, H), lambda b: (0, 0)),
        pl.BlockSpec((1, H, W), lambda b: (b, 0, 0)),
    ],
    out_specs=pl.BlockSpec((1, 3, H, W), lambda b: (b, 0, 0, 0)),
    out_shape=jax.ShapeDtypeStruct((B, 3, H, W), jnp.float32),
)


def kernel(V_matrix, P_matrix, raw_base_points):
    V16 = V_matrix.reshape(B, 16)
    P16 = P_matrix.reshape(B, 16)
    ptsT = jnp.zeros((8, NPAD), jnp.float32)
    ptsT = ptsT.at[0:3, 0:N].set(raw_base_points[:, 0:3].T)
    idx = _proj_call(V16, P16, ptsT)
    raster = _sc_scatter_call()(idx)
    bo = jnp.asarray(_band_ones())
    gg = jnp.asarray(_gauss_op())
    img = _morph_call(bo, gg, gg.T, raster)
    return img


# speedup vs baseline: 1.6063x; 1.4450x over previous
import functools

import numpy as np
import jax
import jax.numpy as jnp
from jax import lax
from jax.experimental import pallas as pl
from jax.experimental.pallas import tpu as pltpu
from jax.experimental.pallas import tpu_sc as plsc

B = 64
N = 13860
H = W = 512
NPAD = 14336
NB = 2048
HW = H * W
NQ = 4
QSIZE = HW // NQ
SENT = -(1 << 20)
NC, NS = 2, 16
NWORK = NC * NS
IMGS_PER_WORKER = B // NWORK

def _bf16_round(x):
    u = lax.bitcast_convert_type(x, jnp.uint32)
    r = (u + jnp.uint32(0x7FFF) + ((u >> 16) & jnp.uint32(1))) & jnp.uint32(0xFFFF0000)
    return lax.bitcast_convert_type(r, jnp.float32)


def _proj_body(v_ref, p_ref, pts_ref, idx_ref):
    j = pl.program_id(0)
    Vm = _bf16_round(v_ref[:])
    Pm = _bf16_round(p_ref[:])
    rows = []
    for i in (0, 1, 3):
        acc = None
        for k in range(4):
            term = Pm[:, 4 * i + k:4 * i + k + 1] * Vm[:, 4 * k:4 * k + 4]
            acc = term if acc is None else acc + term
        rows.append(_bf16_round(acc))
    vpx, vpy, vpw = rows
    px = _bf16_round(pts_ref[0:1, :])
    py = _bf16_round(pts_ref[1:2, :])
    pz = _bf16_round(pts_ref[2:3, :])

    def proj(c):
        return c[:, 0:1] * px + c[:, 1:2] * py + c[:, 2:3] * pz + c[:, 3:4]

    tx = proj(vpx)
    ty = proj(vpy)
    tw = proj(vpw)
    nz = tw != 0.0
    X = jnp.where(nz, tx / tw, tx)
    Y = jnp.where(nz, ty / tw, ty)
    xs = (X + 1.0) * 0.5 * float(W)
    ys = (1.0 - (Y + 1.0) * 0.5) * float(H)
    xr = jnp.round(xs)
    yr = jnp.round(ys)
    valid = (xr >= 0.0) & (xr <= float(W - 1)) & (yr >= 0.0) & (yr <= float(H - 1))
    col = j * NB + lax.broadcasted_iota(jnp.int32, (B, NB), 1)
    valid = valid & (col < N)
    flat = yr.astype(jnp.int32) * W + xr.astype(jnp.int32)
    idx_ref[:] = jnp.where(valid, flat, SENT)


_proj_call = pl.pallas_call(
    _proj_body,
    grid=(NPAD // NB,),
    in_specs=[
        pl.BlockSpec((B, 16), lambda j: (0, 0)),
        pl.BlockSpec((B, 16), lambda j: (0, 0)),
        pl.BlockSpec((8, NB), lambda j: (0, j)),
    ],
    out_specs=pl.BlockSpec((B, NB), lambda j: (0, j)),
    out_shape=jax.ShapeDtypeStruct((B, NPAD), jnp.int32),
)


QROWS = H // NQ


def _sc_scatter_body(idx_hbm, out_hbm, idx_v, qbuf):
    wid = lax.axis_index("s") * NC + lax.axis_index("c")
    zeros16 = jnp.zeros((16,), jnp.float32)
    v255 = jnp.full((16,), 255.0, jnp.float32)
    for im in range(IMGS_PER_WORKER):
        b = wid * IMGS_PER_WORKER + im
        pltpu.sync_copy(idx_hbm.at[b], idx_v)
        for q in range(NQ):
            lo = q * QSIZE

            @pl.loop(0, QROWS)
            def zero_body(i):
                for cc in range(W // 16):
                    qbuf[i, pl.ds(cc * 16, 16)] = zeros16

            @pl.loop(0, NPAD // 16, unroll=8)
            def scan_body(i, lo=lo):
                v = idx_v[pl.ds(i * 16, 16)]
                u = v - lo
                m = lax.bitcast_convert_type(u, jnp.uint32) < jnp.uint32(QSIZE)
                row = u >> 9
                col = u & (W - 1)
                plsc.store_scatter(qbuf, [row, col], v255, mask=m)

            pltpu.sync_copy(qbuf, out_hbm.at[b, pl.ds(q * QROWS, QROWS)])


@functools.lru_cache(maxsize=1)
def _sc_scatter_call():
    mesh = plsc.VectorSubcoreMesh(
        core_axis_name="c", subcore_axis_name="s",
        num_cores=NC, num_subcores=NS)
    return pl.kernel(
        _sc_scatter_body,
        out_type=jax.ShapeDtypeStruct((B, H, W), jnp.float32),
        mesh=mesh,
        compiler_params=pltpu.CompilerParams(needs_layout_passes=False),
        scratch_types=[
            pltpu.VMEM((NPAD,), jnp.int32),
            pltpu.VMEM((QROWS, W), jnp.float32),
        ],
    )


def _gauss_weights():
    sigma = 0.3 * ((9 - 1) * 0.5 - 1.0) + 0.8
    x = np.arange(9, dtype=np.float32) - 4.0
    k = np.exp(-(x.astype(np.float32) ** 2) / np.float32(2.0 * sigma * sigma))
    k = k.astype(np.float32)
    k = k / k.sum(dtype=np.float32)
    return [float(v) for v in k]


_GW = _gauss_weights()


def _band_ones():
    i = np.arange(H)
    return (np.abs(i[:, None] - i[None, :]) <= 4).astype(np.float32)


def _gauss_op():
    g = np.zeros((H, H), np.float32)
    gw = np.asarray(_GW, np.float32)
    for i in range(H):
        for k in range(9):
            t = i + k - 4
            if t < 0:
                t = -t
            elif t > H - 1:
                t = 2 * (H - 1) - t
            g[i, t] += gw[k]
    return g


def _bf(x):
    return jnp.asarray(x).astype(jnp.bfloat16)


def _morph_body(bo_ref, ghi_ref, glo_ref, ghit_ref, glot_ref, r_ref, o_ref):
    m = r_ref[0]
    bo = bo_ref[:]
    c = jnp.dot(bo, m)
    c2 = jnp.dot(c, bo)
    d = jnp.where(c2 > 0.5, 255.0, 0.0).astype(jnp.bfloat16)
    f32 = jnp.float32
    s1 = (jnp.dot(ghi_ref[:], d, preferred_element_type=f32)
          + jnp.dot(glo_ref[:], d, preferred_element_type=f32))
    s1hi = s1.astype(jnp.bfloat16)
    s1lo = (s1 - s1hi.astype(f32)).astype(jnp.bfloat16)
    s = (jnp.dot(s1hi, ghit_ref[:], preferred_element_type=f32)
         + jnp.dot(s1hi, glot_ref[:], preferred_element_type=f32)
         + jnp.dot(s1lo, ghit_ref[:], preferred_element_type=f32))
    ob = jnp.where(s > 100.0, 1.0, 0.0)
    o_ref[0, 0] = ob
    o_ref[0, 1] = ob
    o_ref[0, 2] = ob


_morph_call = pl.pallas_call(
    _morph_body,
    grid=(B,),
    in_specs=[
        pl.BlockSpec((H, H), lambda b: (0, 0)),
        pl.BlockSpec((H, H), lambda b: (0, 0)),
        pl.BlockSpec((H, H), lambda b: (0, 0)),
        pl.BlockSpec((H, H), lambda b: (0, 0)),
        pl.BlockSpec((H, H), lambda b: (0, 0)),
        pl.BlockSpec((1, H, W), lambda b: (b, 0, 0)),
    ],
    out_specs=pl.BlockSpec((1, 3, H, W), lambda b: (b, 0, 0, 0)),
    out_shape=jax.ShapeDtypeStruct((B, 3, H, W), jnp.float32),
)


def kernel(V_matrix, P_matrix, raw_base_points):
    V16 = V_matrix.reshape(B, 16)
    P16 = P_matrix.reshape(B, 16)
    ptsT = jnp.zeros((8, NPAD), jnp.float32)
    ptsT = ptsT.at[0:3, 0:N].set(raw_base_points[:, 0:3].T)
    idx = _proj_call(V16, P16, ptsT)
    raster = _sc_scatter_call()(idx)
    bo = jnp.asarray(_band_ones())
    import ml_dtypes
    g = _gauss_op()
    ghi_np = g.astype(ml_dtypes.bfloat16)
    glo_np = (g - ghi_np.astype(np.float32)).astype(ml_dtypes.bfloat16)
    img = _morph_call(bo, jnp.asarray(ghi_np), jnp.asarray(glo_np),
                      jnp.asarray(ghi_np.T.copy()), jnp.asarray(glo_np.T.copy()),
                      raster)
    return img


# speedup vs baseline: 1.6075x; 1.0008x over previous
import functools

import numpy as np
import jax
import jax.numpy as jnp
from jax import lax
from jax.experimental import pallas as pl
from jax.experimental.pallas import tpu as pltpu
from jax.experimental.pallas import tpu_sc as plsc

B = 64
N = 13860
H = W = 512
NPAD = 14336
NB = 2048
HW = H * W
NQ = 4
QSIZE = HW // NQ
SENT = -(1 << 20)
NC, NS = 2, 16
NWORK = NC * NS
IMGS_PER_WORKER = B // NWORK

def _bf16_round(x):
    u = lax.bitcast_convert_type(x, jnp.uint32)
    r = (u + jnp.uint32(0x7FFF) + ((u >> 16) & jnp.uint32(1))) & jnp.uint32(0xFFFF0000)
    return lax.bitcast_convert_type(r, jnp.float32)


def _proj_body(v_ref, p_ref, pts_ref, idx_ref):
    j = pl.program_id(0)
    Vm = _bf16_round(v_ref[:])
    Pm = _bf16_round(p_ref[:])
    rows = []
    for i in (0, 1, 3):
        acc = None
        for k in range(4):
            term = Pm[:, 4 * i + k:4 * i + k + 1] * Vm[:, 4 * k:4 * k + 4]
            acc = term if acc is None else acc + term
        rows.append(_bf16_round(acc))
    vpx, vpy, vpw = rows
    px = _bf16_round(pts_ref[0:1, :])
    py = _bf16_round(pts_ref[1:2, :])
    pz = _bf16_round(pts_ref[2:3, :])

    def proj(c):
        return c[:, 0:1] * px + c[:, 1:2] * py + c[:, 2:3] * pz + c[:, 3:4]

    tx = proj(vpx)
    ty = proj(vpy)
    tw = proj(vpw)
    nz = tw != 0.0
    X = jnp.where(nz, tx / tw, tx)
    Y = jnp.where(nz, ty / tw, ty)
    xs = (X + 1.0) * 0.5 * float(W)
    ys = (1.0 - (Y + 1.0) * 0.5) * float(H)
    xr = jnp.round(xs)
    yr = jnp.round(ys)
    valid = (xr >= 0.0) & (xr <= float(W - 1)) & (yr >= 0.0) & (yr <= float(H - 1))
    col = j * NB + lax.broadcasted_iota(jnp.int32, (B, NB), 1)
    valid = valid & (col < N)
    flat = yr.astype(jnp.int32) * W + xr.astype(jnp.int32)
    idx_ref[:] = jnp.where(valid, flat, SENT)


_proj_call = pl.pallas_call(
    _proj_body,
    grid=(NPAD // NB,),
    in_specs=[
        pl.BlockSpec((B, 16), lambda j: (0, 0)),
        pl.BlockSpec((B, 16), lambda j: (0, 0)),
        pl.BlockSpec((8, NB), lambda j: (0, j)),
    ],
    out_specs=pl.BlockSpec((B, NB), lambda j: (0, j)),
    out_shape=jax.ShapeDtypeStruct((B, NPAD), jnp.int32),
)


QROWS = H // NQ


def _sc_scatter_body(idx_hbm, out_hbm, idx_v, qbuf):
    wid = lax.axis_index("s") * NC + lax.axis_index("c")
    zeros16 = jnp.zeros((16,), jnp.float32)
    v255 = jnp.full((16,), 255.0, jnp.float32)
    for im in range(IMGS_PER_WORKER):
        b = wid * IMGS_PER_WORKER + im
        pltpu.sync_copy(idx_hbm.at[b], idx_v)
        for q in range(NQ):
            lo = q * QSIZE

            @pl.loop(0, QROWS)
            def zero_body(i):
                for cc in range(W // 16):
                    qbuf[i, pl.ds(cc * 16, 16)] = zeros16

            @pl.loop(0, NPAD // 16, unroll=8)
            def scan_body(i, lo=lo):
                v = idx_v[pl.ds(i * 16, 16)]
                u = v - lo
                m = lax.bitcast_convert_type(u, jnp.uint32) < jnp.uint32(QSIZE)
                row = u >> 9
                col = u & (W - 1)
                plsc.store_scatter(qbuf, [row, col], v255, mask=m)

            pltpu.sync_copy(qbuf, out_hbm.at[b, pl.ds(q * QROWS, QROWS)])


@functools.lru_cache(maxsize=1)
def _sc_scatter_call():
    mesh = plsc.VectorSubcoreMesh(
        core_axis_name="c", subcore_axis_name="s",
        num_cores=NC, num_subcores=NS)
    return pl.kernel(
        _sc_scatter_body,
        out_type=jax.ShapeDtypeStruct((B, H, W), jnp.float32),
        mesh=mesh,
        compiler_params=pltpu.CompilerParams(needs_layout_passes=False),
        scratch_types=[
            pltpu.VMEM((NPAD,), jnp.int32),
            pltpu.VMEM((QROWS, W), jnp.float32),
        ],
    )


def _gauss_weights():
    sigma = 0.3 * ((9 - 1) * 0.5 - 1.0) + 0.8
    x = np.arange(9, dtype=np.float32) - 4.0
    k = np.exp(-(x.astype(np.float32) ** 2) / np.float32(2.0 * sigma * sigma))
    k = k.astype(np.float32)
    k = k / k.sum(dtype=np.float32)
    return [float(v) for v in k]


_GW = _gauss_weights()


def _band_ones():
    i = np.arange(H)
    return (np.abs(i[:, None] - i[None, :]) <= 4).astype(np.float32)


def _gauss_op():
    g = np.zeros((H, H), np.float32)
    gw = np.asarray(_GW, np.float32)
    for i in range(H):
        for k in range(9):
            t = i + k - 4
            if t < 0:
                t = -t
            elif t > H - 1:
                t = 2 * (H - 1) - t
            g[i, t] += gw[k]
    return g


def _bf(x):
    return jnp.asarray(x).astype(jnp.bfloat16)


def _morph_body(bo_ref, ghi_ref, glo_ref, ghit_ref, glot_ref, r_ref, o_ref):
    m = r_ref[0]
    bo = bo_ref[:]
    f32 = jnp.float32
    mb = jnp.where(m > 0.0, 1.0, 0.0).astype(jnp.bfloat16)
    c = jnp.dot(bo, mb, preferred_element_type=f32).astype(jnp.bfloat16)
    c2 = jnp.dot(c, bo, preferred_element_type=f32)
    d = jnp.where(c2 > 0.5, 255.0, 0.0).astype(jnp.bfloat16)
    f32 = jnp.float32
    s1 = (jnp.dot(ghi_ref[:], d, preferred_element_type=f32)
          + jnp.dot(glo_ref[:], d, preferred_element_type=f32))
    s1hi = s1.astype(jnp.bfloat16)
    s1lo = (s1 - s1hi.astype(f32)).astype(jnp.bfloat16)
    s = (jnp.dot(s1hi, ghit_ref[:], preferred_element_type=f32)
         + jnp.dot(s1hi, glot_ref[:], preferred_element_type=f32)
         + jnp.dot(s1lo, ghit_ref[:], preferred_element_type=f32))
    ob = jnp.where(s > 100.0, 1.0, 0.0)
    o_ref[0, 0] = ob
    o_ref[0, 1] = ob
    o_ref[0, 2] = ob


_morph_call = pl.pallas_call(
    _morph_body,
    grid=(B,),
    in_specs=[
        pl.BlockSpec((H, H), lambda b: (0, 0)),
        pl.BlockSpec((H, H), lambda b: (0, 0)),
        pl.BlockSpec((H, H), lambda b: (0, 0)),
        pl.BlockSpec((H, H), lambda b: (0, 0)),
        pl.BlockSpec((H, H), lambda b: (0, 0)),
        pl.BlockSpec((1, H, W), lambda b: (b, 0, 0)),
    ],
    out_specs=pl.BlockSpec((1, 3, H, W), lambda b: (b, 0, 0, 0)),
    out_shape=jax.ShapeDtypeStruct((B, 3, H, W), jnp.float32),
)


def kernel(V_matrix, P_matrix, raw_base_points):
    V16 = V_matrix.reshape(B, 16)
    P16 = P_matrix.reshape(B, 16)
    ptsT = jnp.zeros((8, NPAD), jnp.float32)
    ptsT = ptsT.at[0:3, 0:N].set(raw_base_points[:, 0:3].T)
    idx = _proj_call(V16, P16, ptsT)
    raster = _sc_scatter_call()(idx)
    import ml_dtypes
    bo = jnp.asarray(_band_ones().astype(ml_dtypes.bfloat16))
    g = _gauss_op()
    ghi_np = g.astype(ml_dtypes.bfloat16)
    glo_np = (g - ghi_np.astype(np.float32)).astype(ml_dtypes.bfloat16)
    img = _morph_call(bo, jnp.asarray(ghi_np), jnp.asarray(glo_np),
                      jnp.asarray(ghi_np.T.copy()), jnp.asarray(glo_np.T.copy()),
                      raster)
    return img


# speedup vs baseline: 1.8933x; 1.1778x over previous
import functools

import numpy as np
import jax
import jax.numpy as jnp
from jax import lax
from jax.experimental import pallas as pl
from jax.experimental.pallas import tpu as pltpu
from jax.experimental.pallas import tpu_sc as plsc

B = 64
N = 13860
H = W = 512
NPAD = 14336
NB = 2048
HW = H * W
NQ = 4
QSIZE = HW // NQ
SENT = -(1 << 20)
NC, NS = 2, 16
NWORK = NC * NS
IMGS_PER_WORKER = B // NWORK

def _bf16_round(x):
    u = lax.bitcast_convert_type(x, jnp.uint32)
    r = (u + jnp.uint32(0x7FFF) + ((u >> 16) & jnp.uint32(1))) & jnp.uint32(0xFFFF0000)
    return lax.bitcast_convert_type(r, jnp.float32)


def _proj_body(v_ref, p_ref, pts_ref, idx_ref):
    j = pl.program_id(0)
    Vm = _bf16_round(v_ref[:])
    Pm = _bf16_round(p_ref[:])
    rows = []
    for i in (0, 1, 3):
        acc = None
        for k in range(4):
            term = Pm[:, 4 * i + k:4 * i + k + 1] * Vm[:, 4 * k:4 * k + 4]
            acc = term if acc is None else acc + term
        rows.append(_bf16_round(acc))
    vpx, vpy, vpw = rows
    px = _bf16_round(pts_ref[0:1, :])
    py = _bf16_round(pts_ref[1:2, :])
    pz = _bf16_round(pts_ref[2:3, :])

    def proj(c):
        return c[:, 0:1] * px + c[:, 1:2] * py + c[:, 2:3] * pz + c[:, 3:4]

    tx = proj(vpx)
    ty = proj(vpy)
    tw = proj(vpw)
    nz = tw != 0.0
    X = jnp.where(nz, tx / tw, tx)
    Y = jnp.where(nz, ty / tw, ty)
    xs = (X + 1.0) * 0.5 * float(W)
    ys = (1.0 - (Y + 1.0) * 0.5) * float(H)
    xr = jnp.round(xs)
    yr = jnp.round(ys)
    valid = (xr >= 0.0) & (xr <= float(W - 1)) & (yr >= 0.0) & (yr <= float(H - 1))
    col = j * NB + lax.broadcasted_iota(jnp.int32, (B, NB), 1)
    valid = valid & (col < N)
    flat = yr.astype(jnp.int32) * W + xr.astype(jnp.int32)
    idx_ref[:] = jnp.where(valid, flat, SENT)


_proj_call = pl.pallas_call(
    _proj_body,
    grid=(NPAD // NB,),
    in_specs=[
        pl.BlockSpec((B, 16), lambda j: (0, 0)),
        pl.BlockSpec((B, 16), lambda j: (0, 0)),
        pl.BlockSpec((8, NB), lambda j: (0, j)),
    ],
    out_specs=pl.BlockSpec((B, NB), lambda j: (0, j)),
    out_shape=jax.ShapeDtypeStruct((B, NPAD), jnp.int32),
)


QROWS = H // NQ


HB = B // 2


def _make_sc_body(base):
    def _sc_scatter_body(idx_hbm, out_hbm, idx_v, qbuf):
        wid = lax.axis_index("s") * NC + lax.axis_index("c")
        zeros16 = jnp.zeros((16,), jnp.float32)
        v255 = jnp.full((16,), 255.0, jnp.float32)
        pltpu.sync_copy(idx_hbm.at[base + wid], idx_v)
        for q in range(NQ):
            lo = q * QSIZE

            @pl.loop(0, QROWS)
            def zero_body(i):
                for cc in range(W // 16):
                    qbuf[i, pl.ds(cc * 16, 16)] = zeros16

            @pl.loop(0, NPAD // 16, unroll=8)
            def scan_body(i, lo=lo):
                v = idx_v[pl.ds(i * 16, 16)]
                u = v - lo
                m = lax.bitcast_convert_type(u, jnp.uint32) < jnp.uint32(QSIZE)
                row = u >> 9
                col = u & (W - 1)
                plsc.store_scatter(qbuf, [row, col], v255, mask=m)

            pltpu.sync_copy(qbuf, out_hbm.at[wid, pl.ds(q * QROWS, QROWS)])

    return _sc_scatter_body


@functools.lru_cache(maxsize=2)
def _sc_scatter_call(base):
    mesh = plsc.VectorSubcoreMesh(
        core_axis_name="c", subcore_axis_name="s",
        num_cores=NC, num_subcores=NS)
    return pl.kernel(
        _make_sc_body(base),
        out_type=jax.ShapeDtypeStruct((HB, H, W), jnp.float32),
        mesh=mesh,
        compiler_params=pltpu.CompilerParams(needs_layout_passes=False),
        scratch_types=[
            pltpu.VMEM((NPAD,), jnp.int32),
            pltpu.VMEM((QROWS, W), jnp.float32),
        ],
    )


def _gauss_weights():
    sigma = 0.3 * ((9 - 1) * 0.5 - 1.0) + 0.8
    x = np.arange(9, dtype=np.float32) - 4.0
    k = np.exp(-(x.astype(np.float32) ** 2) / np.float32(2.0 * sigma * sigma))
    k = k.astype(np.float32)
    k = k / k.sum(dtype=np.float32)
    return [float(v) for v in k]


_GW = _gauss_weights()


def _band_ones():
    i = np.arange(H)
    return (np.abs(i[:, None] - i[None, :]) <= 4).astype(np.float32)


def _gauss_op():
    g = np.zeros((H, H), np.float32)
    gw = np.asarray(_GW, np.float32)
    for i in range(H):
        for k in range(9):
            t = i + k - 4
            if t < 0:
                t = -t
            elif t > H - 1:
                t = 2 * (H - 1) - t
            g[i, t] += gw[k]
    return g


def _bf(x):
    return jnp.asarray(x).astype(jnp.bfloat16)


def _morph_body(bo_ref, ghi_ref, glo_ref, ghit_ref, glot_ref, r_ref, o_ref):
    m = r_ref[0]
    bo = bo_ref[:]
    f32 = jnp.float32
    mb = jnp.where(m > 0.0, 1.0, 0.0).astype(jnp.bfloat16)
    c = jnp.dot(bo, mb, preferred_element_type=f32).astype(jnp.bfloat16)
    c2 = jnp.dot(c, bo, preferred_element_type=f32)
    d = jnp.where(c2 > 0.5, 255.0, 0.0).astype(jnp.bfloat16)
    f32 = jnp.float32
    s1 = (jnp.dot(ghi_ref[:], d, preferred_element_type=f32)
          + jnp.dot(glo_ref[:], d, preferred_element_type=f32))
    s1hi = s1.astype(jnp.bfloat16)
    s1lo = (s1 - s1hi.astype(f32)).astype(jnp.bfloat16)
    s = (jnp.dot(s1hi, ghit_ref[:], preferred_element_type=f32)
         + jnp.dot(s1hi, glot_ref[:], preferred_element_type=f32)
         + jnp.dot(s1lo, ghit_ref[:], preferred_element_type=f32))
    ob = jnp.where(s > 100.0, 1.0, 0.0)
    o_ref[0, 0] = ob
    o_ref[0, 1] = ob
    o_ref[0, 2] = ob


_OP_SPECS = [pl.BlockSpec((H, H), lambda b: (0, 0)) for _ in range(5)]

_morph_call_a = pl.pallas_call(
    _morph_body,
    grid=(HB,),
    in_specs=_OP_SPECS + [pl.BlockSpec((1, H, W), lambda b: (b, 0, 0))],
    out_specs=pl.BlockSpec((1, 3, H, W), lambda b: (b, 0, 0, 0)),
    out_shape=jax.ShapeDtypeStruct((B, 3, H, W), jnp.float32),
)


def _morph_body_b(bo_ref, ghi_ref, glo_ref, ghit_ref, glot_ref, r_ref,
                  prev_ref, o_ref):
    del prev_ref
    _morph_body(bo_ref, ghi_ref, glo_ref, ghit_ref, glot_ref, r_ref, o_ref)


_morph_call_b = pl.pallas_call(
    _morph_body_b,
    grid=(HB,),
    in_specs=_OP_SPECS + [
        pl.BlockSpec((1, H, W), lambda b: (b, 0, 0)),
        pl.BlockSpec(memory_space=pl.ANY),
    ],
    out_specs=pl.BlockSpec((1, 3, H, W), lambda b: (b + HB, 0, 0, 0)),
    out_shape=jax.ShapeDtypeStruct((B, 3, H, W), jnp.float32),
    input_output_aliases={6: 0},
)


def kernel(V_matrix, P_matrix, raw_base_points):
    V16 = V_matrix.reshape(B, 16)
    P16 = P_matrix.reshape(B, 16)
    ptsT = jnp.zeros((8, NPAD), jnp.float32)
    ptsT = ptsT.at[0:3, 0:N].set(raw_base_points[:, 0:3].T)
    idx = _proj_call(V16, P16, ptsT)
    rast_a = _sc_scatter_call(0)(idx)
    rast_b = _sc_scatter_call(HB)(idx)
    import ml_dtypes
    bo = jnp.asarray(_band_ones().astype(ml_dtypes.bfloat16))
    g = _gauss_op()
    ghi_np = g.astype(ml_dtypes.bfloat16)
    glo_np = (g - ghi_np.astype(np.float32)).astype(ml_dtypes.bfloat16)
    ghi = jnp.asarray(ghi_np)
    glo = jnp.asarray(glo_np)
    ghit = jnp.asarray(ghi_np.T.copy())
    glot = jnp.asarray(glo_np.T.copy())
    img_a = _morph_call_a(bo, ghi, glo, ghit, glot, rast_a)
    img = _morph_call_b(bo, ghi, glo, ghit, glot, rast_b, img_a)
    return img


# speedup vs baseline: 2.0118x; 1.0626x over previous
import functools

import numpy as np
import jax
import jax.numpy as jnp
from jax import lax
from jax.experimental import pallas as pl
from jax.experimental.pallas import tpu as pltpu
from jax.experimental.pallas import tpu_sc as plsc

B = 64
N = 13860
H = W = 512
NPAD = 14336
NB = 2048
HW = H * W
NQ = 4
QSIZE = HW // NQ
SENT = -(1 << 20)
NC, NS = 2, 16
NWORK = NC * NS
IMGS_PER_WORKER = B // NWORK

def _bf16_round(x):
    u = lax.bitcast_convert_type(x, jnp.uint32)
    r = (u + jnp.uint32(0x7FFF) + ((u >> 16) & jnp.uint32(1))) & jnp.uint32(0xFFFF0000)
    return lax.bitcast_convert_type(r, jnp.float32)


def _proj_body(v_ref, p_ref, pts_ref, idx_ref):
    j = pl.program_id(0)
    Vm = _bf16_round(v_ref[:])
    Pm = _bf16_round(p_ref[:])
    rows = []
    for i in (0, 1, 3):
        acc = None
        for k in range(4):
            term = Pm[:, 4 * i + k:4 * i + k + 1] * Vm[:, 4 * k:4 * k + 4]
            acc = term if acc is None else acc + term
        rows.append(_bf16_round(acc))
    vpx, vpy, vpw = rows
    px = _bf16_round(pts_ref[0:1, :])
    py = _bf16_round(pts_ref[1:2, :])
    pz = _bf16_round(pts_ref[2:3, :])

    def proj(c):
        return c[:, 0:1] * px + c[:, 1:2] * py + c[:, 2:3] * pz + c[:, 3:4]

    tx = proj(vpx)
    ty = proj(vpy)
    tw = proj(vpw)
    nz = tw != 0.0
    X = jnp.where(nz, tx / tw, tx)
    Y = jnp.where(nz, ty / tw, ty)
    xs = (X + 1.0) * 0.5 * float(W)
    ys = (1.0 - (Y + 1.0) * 0.5) * float(H)
    xr = jnp.round(xs)
    yr = jnp.round(ys)
    valid = (xr >= 0.0) & (xr <= float(W - 1)) & (yr >= 0.0) & (yr <= float(H - 1))
    col = j * NB + lax.broadcasted_iota(jnp.int32, (B, NB), 1)
    valid = valid & (col < N)
    flat = yr.astype(jnp.int32) * W + xr.astype(jnp.int32)
    idx_ref[:] = jnp.where(valid, flat, SENT)


_proj_call = pl.pallas_call(
    _proj_body,
    grid=(NPAD // NB,),
    in_specs=[
        pl.BlockSpec((B, 16), lambda j: (0, 0)),
        pl.BlockSpec((B, 16), lambda j: (0, 0)),
        pl.BlockSpec((8, NB), lambda j: (0, j)),
    ],
    out_specs=pl.BlockSpec((B, NB), lambda j: (0, j)),
    out_shape=jax.ShapeDtypeStruct((B, NPAD), jnp.int32),
)


QROWS = H // NQ


NSPLIT = 4
SEG = B // NSPLIT


def _make_sc_body(base):
    tpw = SEG * NQ // NWORK

    def _do_quarter(idx_v, qbuf, out_hbm, bl, q, lo):
        zeros16 = jnp.zeros((16,), jnp.float32)
        v255 = jnp.full((16,), 255.0, jnp.float32)

        @pl.loop(0, QROWS)
        def zero_body(i):
            for cc in range(W // 16):
                qbuf[i, pl.ds(cc * 16, 16)] = zeros16

        @pl.loop(0, NPAD // 16, unroll=8)
        def scan_body(i):
            v = idx_v[pl.ds(i * 16, 16)]
            u = v - lo
            m = lax.bitcast_convert_type(u, jnp.uint32) < jnp.uint32(QSIZE)
            row = u >> 9
            col = u & (W - 1)
            plsc.store_scatter(qbuf, [row, col], v255, mask=m)

        pltpu.sync_copy(qbuf, out_hbm.at[bl, pl.ds(q * QROWS, QROWS)])

    def _sc_scatter_body(idx_hbm, out_hbm, idx_v, qbuf):
        wid = lax.axis_index("s") * NC + lax.axis_index("c")
        if tpw >= NQ:
            for im in range(tpw // NQ):
                bl = wid * (tpw // NQ) + im
                pltpu.sync_copy(idx_hbm.at[base + bl], idx_v)
                for q in range(NQ):
                    _do_quarter(idx_v, qbuf, out_hbm, bl, q, q * QSIZE)
        else:
            nfrac = NQ // tpw
            bl = wid // nfrac
            q0 = (wid % nfrac) * tpw
            pltpu.sync_copy(idx_hbm.at[base + bl], idx_v)
            for t in range(tpw):
                q = q0 + t
                _do_quarter(idx_v, qbuf, out_hbm, bl, q, q * QSIZE)

    return _sc_scatter_body


@functools.lru_cache(maxsize=NSPLIT)
def _sc_scatter_call(base):
    mesh = plsc.VectorSubcoreMesh(
        core_axis_name="c", subcore_axis_name="s",
        num_cores=NC, num_subcores=NS)
    return pl.kernel(
        _make_sc_body(base),
        out_type=jax.ShapeDtypeStruct((SEG, H, W), jnp.float32),
        mesh=mesh,
        compiler_params=pltpu.CompilerParams(needs_layout_passes=False),
        scratch_types=[
            pltpu.VMEM((NPAD,), jnp.int32),
            pltpu.VMEM((QROWS, W), jnp.float32),
        ],
    )


def _gauss_weights():
    sigma = 0.3 * ((9 - 1) * 0.5 - 1.0) + 0.8
    x = np.arange(9, dtype=np.float32) - 4.0
    k = np.exp(-(x.astype(np.float32) ** 2) / np.float32(2.0 * sigma * sigma))
    k = k.astype(np.float32)
    k = k / k.sum(dtype=np.float32)
    return [float(v) for v in k]


_GW = _gauss_weights()


def _band_ones():
    i = np.arange(H)
    return (np.abs(i[:, None] - i[None, :]) <= 4).astype(np.float32)


def _gauss_op():
    g = np.zeros((H, H), np.float32)
    gw = np.asarray(_GW, np.float32)
    for i in range(H):
        for k in range(9):
            t = i + k - 4
            if t < 0:
                t = -t
            elif t > H - 1:
                t = 2 * (H - 1) - t
            g[i, t] += gw[k]
    return g


def _bf(x):
    return jnp.asarray(x).astype(jnp.bfloat16)


def _morph_body(bo_ref, ghi_ref, glo_ref, ghit_ref, glot_ref, r_ref, o_ref):
    m = r_ref[0]
    bo = bo_ref[:]
    f32 = jnp.float32
    mb = jnp.where(m > 0.0, 1.0, 0.0).astype(jnp.bfloat16)
    c = jnp.dot(bo, mb, preferred_element_type=f32).astype(jnp.bfloat16)
    c2 = jnp.dot(c, bo, preferred_element_type=f32)
    d = jnp.where(c2 > 0.5, 255.0, 0.0).astype(jnp.bfloat16)
    f32 = jnp.float32
    s1 = (jnp.dot(ghi_ref[:], d, preferred_element_type=f32)
          + jnp.dot(glo_ref[:], d, preferred_element_type=f32))
    s1hi = s1.astype(jnp.bfloat16)
    s1lo = (s1 - s1hi.astype(f32)).astype(jnp.bfloat16)
    s = (jnp.dot(s1hi, ghit_ref[:], preferred_element_type=f32)
         + jnp.dot(s1hi, glot_ref[:], preferred_element_type=f32)
         + jnp.dot(s1lo, ghit_ref[:], preferred_element_type=f32))
    ob = jnp.where(s > 100.0, 1.0, 0.0)
    o_ref[0, 0] = ob
    o_ref[0, 1] = ob
    o_ref[0, 2] = ob


_OP_SPECS = [pl.BlockSpec((H, H), lambda b: (0, 0)) for _ in range(5)]


def _morph_body_b(bo_ref, ghi_ref, glo_ref, ghit_ref, glot_ref, r_ref,
                  prev_ref, o_ref):
    del prev_ref
    _morph_body(bo_ref, ghi_ref, glo_ref, ghit_ref, glot_ref, r_ref, o_ref)


def _make_morph_call(base, aliased):
    in_specs = list(_OP_SPECS) + [pl.BlockSpec((1, H, W), lambda b: (b, 0, 0))]
    kwargs = {}
    if aliased:
        in_specs.append(pl.BlockSpec(memory_space=pl.ANY))
        kwargs["input_output_aliases"] = {6: 0}
    return pl.pallas_call(
        _morph_body_b if aliased else _morph_body,
        grid=(SEG,),
        in_specs=in_specs,
        out_specs=pl.BlockSpec(
            (1, 3, H, W), lambda b, base=base: (b + base, 0, 0, 0)),
        out_shape=jax.ShapeDtypeStruct((B, 3, H, W), jnp.float32),
        **kwargs,
    )


def kernel(V_matrix, P_matrix, raw_base_points):
    V16 = V_matrix.reshape(B, 16)
    P16 = P_matrix.reshape(B, 16)
    ptsT = jnp.zeros((8, NPAD), jnp.float32)
    ptsT = ptsT.at[0:3, 0:N].set(raw_base_points[:, 0:3].T)
    idx = _proj_call(V16, P16, ptsT)
    rasts = [_sc_scatter_call(k * SEG)(idx) for k in range(NSPLIT)]
    import ml_dtypes
    bo = jnp.asarray(_band_ones().astype(ml_dtypes.bfloat16))
    g = _gauss_op()
    ghi_np = g.astype(ml_dtypes.bfloat16)
    glo_np = (g - ghi_np.astype(np.float32)).astype(ml_dtypes.bfloat16)
    ghi = jnp.asarray(ghi_np)
    glo = jnp.asarray(glo_np)
    ghit = jnp.asarray(ghi_np.T.copy())
    glot = jnp.asarray(glo_np.T.copy())
    img = _make_morph_call(0, False)(bo, ghi, glo, ghit, glot, rasts[0])
    for k in range(1, NSPLIT):
        img = _make_morph_call(k * SEG, True)(
            bo, ghi, glo, ghit, glot, rasts[k], img)
    return img
